# Initial kernel scaffold; baseline (speedup 1.0000x reference)
import functools

import jax
import jax.numpy as jnp
from jax import lax
from jax.experimental import pallas as pl
from jax.experimental.pallas import tpu as pltpu
from jax.experimental.pallas import tpu_sc as plsc

HIDDEN = 2048
NUM_EXPERTS = 16
TOP_K = 2


def _logits_body(x_ref, w_ref, out_ref):
    out_ref[...] = lax.dot_general(
        x_ref[...],
        w_ref[...],
        dimension_numbers=(((1,), (1,)), ((), ())),
        preferred_element_type=jnp.float32,
    )


def _logits(xf, W):
    n = xf.shape[0]
    bt = 2048
    return pl.pallas_call(
        _logits_body,
        grid=(n // bt,),
        in_specs=[
            pl.BlockSpec((bt, HIDDEN), lambda i: (i, 0)),
            pl.BlockSpec((NUM_EXPERTS, HIDDEN), lambda i: (0, 0)),
        ],
        out_specs=pl.BlockSpec((bt, NUM_EXPERTS), lambda i: (i, 0)),
        out_shape=jax.ShapeDtypeStruct((n, NUM_EXPERTS), jnp.float32),
    )(xf, W)


def _router(logits):
    n = logits.shape[0]
    info = plsc.get_sparse_core_info()
    nw = info.num_cores * info.num_subcores
    tpw = n // nw
    mesh = plsc.VectorSubcoreMesh(core_axis_name="c", subcore_axis_name="s")

    @functools.partial(
        pl.kernel,
        out_type=(
            jax.ShapeDtypeStruct((n, NUM_EXPERTS), jnp.float32),
            jax.ShapeDtypeStruct((n, TOP_K), jnp.float32),
            jax.ShapeDtypeStruct((n, TOP_K), jnp.int32),
        ),
        mesh=mesh,
        scratch_types=[
            pltpu.VMEM((tpw, NUM_EXPERTS), jnp.float32),
            pltpu.VMEM((tpw, NUM_EXPERTS), jnp.float32),
            pltpu.VMEM((tpw, TOP_K), jnp.float32),
            pltpu.VMEM((tpw, TOP_K), jnp.int32),
        ],
    )
    def run(logits_hbm, scores_hbm, weights_hbm, inds_hbm, lg_v, sc_v, w_v, i_v):
        wid = lax.axis_index("s") * info.num_cores + lax.axis_index("c")
        base = wid * tpw
        pltpu.sync_copy(logits_hbm.at[pl.ds(base, tpw)], lg_v)
        lanes = lax.iota(jnp.int32, NUM_EXPERTS)

        def body(t, carry):
            row = lg_v[t]
            m1 = jnp.max(row)
            i1 = plsc.all_reduce_ffs(row == m1)
            sel1 = lanes == i1
            row2 = jnp.where(sel1, -jnp.inf, row)
            m2 = jnp.max(row2)
            i2 = plsc.all_reduce_ffs(row2 == m2)
            sel2 = lanes == i2
            chosen = sel1 | sel2
            sig = 1.0 / (1.0 + jnp.exp(-row))
            sc_v[t] = jnp.where(chosen, sig, 0.0)
            rowv = jnp.full((NUM_EXPERTS,), t, jnp.int32)
            colv = jnp.where(sel1, 0, 1)
            plsc.store_scatter(w_v, [rowv, colv], sig, mask=chosen)
            plsc.store_scatter(i_v, [rowv, colv], lanes, mask=chosen)
            return carry

        lax.fori_loop(0, tpw, body, 0)
        pltpu.sync_copy(sc_v, scores_hbm.at[pl.ds(base, tpw)])
        pltpu.sync_copy(w_v, weights_hbm.at[pl.ds(base, tpw)])
        pltpu.sync_copy(i_v, inds_hbm.at[pl.ds(base, tpw)])

    return run(logits)


def kernel(x, W):
    xf = x.reshape(-1, x.shape[-1])
    logits = _logits(xf, W)
    scores, weights, inds = _router(logits)
    return (scores, weights, inds)


# speedup vs baseline: 1.8619x; 1.8619x over previous
import functools

import jax
import jax.numpy as jnp
from jax import lax
from jax.experimental import pallas as pl
from jax.experimental.pallas import tpu as pltpu
from jax.experimental.pallas import tpu_sc as plsc

HIDDEN = 2048
NUM_EXPERTS = 16
TOP_K = 2


def _logits_body(x_ref, w_ref, out_ref):
    out_ref[...] = lax.dot_general(
        x_ref[...],
        w_ref[...],
        dimension_numbers=(((1,), (1,)), ((), ())),
        preferred_element_type=jnp.float32,
    )


def _logits(xf, W):
    n = xf.shape[0]
    bt = 2048
    return pl.pallas_call(
        _logits_body,
        grid=(n // bt,),
        in_specs=[
            pl.BlockSpec((bt, HIDDEN), lambda i: (i, 0)),
            pl.BlockSpec((NUM_EXPERTS, HIDDEN), lambda i: (0, 0)),
        ],
        out_specs=pl.BlockSpec((bt, NUM_EXPERTS), lambda i: (i, 0)),
        out_shape=jax.ShapeDtypeStruct((n, NUM_EXPERTS), jnp.float32),
    )(xf, W)


def _router(logits_flat, n):
    info = plsc.get_sparse_core_info()
    nw = info.num_cores * info.num_subcores
    tpw = n // nw
    mesh = plsc.VectorSubcoreMesh(core_axis_name="c", subcore_axis_name="s")

    @functools.partial(
        pl.kernel,
        out_type=(
            jax.ShapeDtypeStruct((n * NUM_EXPERTS,), jnp.float32),
            jax.ShapeDtypeStruct((n * TOP_K,), jnp.float32),
            jax.ShapeDtypeStruct((n * TOP_K,), jnp.int32),
        ),
        mesh=mesh,
        compiler_params=pltpu.CompilerParams(needs_layout_passes=False),
        scratch_types=[
            pltpu.VMEM((tpw * NUM_EXPERTS,), jnp.float32),
            pltpu.VMEM((tpw * NUM_EXPERTS,), jnp.float32),
            pltpu.VMEM((tpw * TOP_K,), jnp.float32),
            pltpu.VMEM((tpw * TOP_K,), jnp.int32),
        ],
    )
    def run(logits_hbm, scores_hbm, weights_hbm, inds_hbm, lg_v, sc_v, w_v, i_v):
        wid = lax.axis_index("s") * info.num_cores + lax.axis_index("c")
        base = wid * tpw
        pltpu.sync_copy(
            logits_hbm.at[pl.ds(base * NUM_EXPERTS, tpw * NUM_EXPERTS)], lg_v
        )
        lanes = lax.iota(jnp.int32, NUM_EXPERTS)

        def argmax_tree(vals):
            v, i = vals, lanes
            for s in (1, 2, 4, 8):
                perm = lanes ^ s
                ov = v.at[perm].get(mode="promise_in_bounds")
                oi = i.at[perm].get(mode="promise_in_bounds")
                take = (ov > v) | ((ov == v) & (oi < i))
                v = jnp.where(take, ov, v)
                i = jnp.where(take, oi, i)
            return v, i

        def body(t, carry):
            off = pl.multiple_of(t * NUM_EXPERTS, NUM_EXPERTS)
            row = lg_v[pl.ds(off, NUM_EXPERTS)]
            _, i1 = argmax_tree(row)
            sel1 = lanes == i1
            row2 = jnp.where(sel1, -jnp.inf, row)
            _, i2 = argmax_tree(row2)
            sel2 = lanes == i2
            chosen = sel1 | sel2
            sig = 1.0 / (1.0 + jnp.exp(-row))
            sc_v[pl.ds(off, NUM_EXPERTS)] = jnp.where(chosen, sig, 0.0)
            flat = jnp.where(sel1, TOP_K * t, TOP_K * t + 1)
            plsc.store_scatter(w_v, [flat], sig, mask=chosen)
            plsc.store_scatter(i_v, [flat], lanes, mask=chosen)
            return carry

        lax.fori_loop(0, tpw, body, 0)
        pltpu.sync_copy(
            sc_v, scores_hbm.at[pl.ds(base * NUM_EXPERTS, tpw * NUM_EXPERTS)]
        )
        pltpu.sync_copy(w_v, weights_hbm.at[pl.ds(base * TOP_K, tpw * TOP_K)])
        pltpu.sync_copy(i_v, inds_hbm.at[pl.ds(base * TOP_K, tpw * TOP_K)])

    return run(logits_flat)


def kernel(x, W):
    xf = x.reshape(-1, x.shape[-1])
    n = xf.shape[0]
    logits = _logits(xf, W)
    scores, weights, inds = _router(logits.reshape(-1), n)
    return (
        scores.reshape(n, NUM_EXPERTS),
        weights.reshape(n, TOP_K),
        inds.reshape(n, TOP_K),
    )


# speedup vs baseline: 1.9392x; 1.0415x over previous
import functools

import jax
import jax.numpy as jnp
from jax import lax
from jax.experimental import pallas as pl
from jax.experimental.pallas import tpu as pltpu
from jax.experimental.pallas import tpu_sc as plsc

HIDDEN = 2048
NUM_EXPERTS = 16
TOP_K = 2


def _logits_body(x_ref, w_ref, out_ref):
    out_ref[...] = lax.dot_general(
        x_ref[...],
        w_ref[...],
        dimension_numbers=(((1,), (1,)), ((), ())),
        preferred_element_type=jnp.float32,
    )


def _logits(xf, W):
    n = xf.shape[0]
    bt = 2048
    return pl.pallas_call(
        _logits_body,
        grid=(n // bt,),
        in_specs=[
            pl.BlockSpec((bt, HIDDEN), lambda i: (i, 0)),
            pl.BlockSpec((NUM_EXPERTS, HIDDEN), lambda i: (0, 0)),
        ],
        out_specs=pl.BlockSpec((bt, NUM_EXPERTS), lambda i: (i, 0)),
        out_shape=jax.ShapeDtypeStruct((n, NUM_EXPERTS), jnp.float32),
    )(xf, W)


def _router(logits_flat, n):
    info = plsc.get_sparse_core_info()
    nw = info.num_cores * info.num_subcores
    tpw = n // nw
    mesh = plsc.VectorSubcoreMesh(core_axis_name="c", subcore_axis_name="s")

    @functools.partial(
        pl.kernel,
        out_type=(
            jax.ShapeDtypeStruct((n * NUM_EXPERTS,), jnp.float32),
            jax.ShapeDtypeStruct((n * TOP_K,), jnp.float32),
            jax.ShapeDtypeStruct((n * TOP_K,), jnp.int32),
        ),
        mesh=mesh,
        compiler_params=pltpu.CompilerParams(needs_layout_passes=False),
        scratch_types=[
            pltpu.VMEM((tpw * NUM_EXPERTS,), jnp.float32),
            pltpu.VMEM((tpw * NUM_EXPERTS,), jnp.float32),
            pltpu.VMEM((tpw * TOP_K,), jnp.float32),
            pltpu.VMEM((tpw * TOP_K,), jnp.int32),
        ],
    )
    def run(logits_hbm, scores_hbm, weights_hbm, inds_hbm, lg_v, sc_v, w_v, i_v):
        wid = lax.axis_index("s") * info.num_cores + lax.axis_index("c")
        base = wid * tpw
        pltpu.sync_copy(
            logits_hbm.at[pl.ds(base * NUM_EXPERTS, tpw * NUM_EXPERTS)], lg_v
        )
        lanes = lax.iota(jnp.int32, NUM_EXPERTS)

        def max_tree(v):
            for s in (1, 2, 4, 8):
                v = jnp.maximum(v, v.at[lanes ^ s].get(mode="promise_in_bounds"))
            return v

        def body(t, carry):
            off = pl.multiple_of(t * NUM_EXPERTS, NUM_EXPERTS)
            row = lg_v[pl.ds(off, NUM_EXPERTS)]
            m1 = max_tree(row)
            i1 = plsc.all_reduce_ffs(row == m1)
            sel1 = lanes == i1
            row2 = jnp.where(sel1, -jnp.inf, row)
            m2 = max_tree(row2)
            i2 = plsc.all_reduce_ffs(row2 == m2)
            sel2 = lanes == i2
            chosen = sel1 | sel2
            sig = 1.0 / (1.0 + jnp.exp(-row))
            sc_v[pl.ds(off, NUM_EXPERTS)] = jnp.where(chosen, sig, 0.0)
            flat = jnp.where(sel1, TOP_K * t, TOP_K * t + 1)
            plsc.store_scatter(w_v, [flat], sig, mask=chosen)
            plsc.store_scatter(i_v, [flat], lanes, mask=chosen)
            return carry

        lax.fori_loop(0, tpw, body, 0)
        pltpu.sync_copy(
            sc_v, scores_hbm.at[pl.ds(base * NUM_EXPERTS, tpw * NUM_EXPERTS)]
        )
        pltpu.sync_copy(w_v, weights_hbm.at[pl.ds(base * TOP_K, tpw * TOP_K)])
        pltpu.sync_copy(i_v, inds_hbm.at[pl.ds(base * TOP_K, tpw * TOP_K)])

    return run(logits_flat)


def kernel(x, W):
    xf = x.reshape(-1, x.shape[-1])
    n = xf.shape[0]
    logits = _logits(xf, W)
    scores, weights, inds = _router(logits.reshape(-1), n)
    return (
        scores.reshape(n, NUM_EXPERTS),
        weights.reshape(n, TOP_K),
        inds.reshape(n, TOP_K),
    )


# speedup vs baseline: 2.0993x; 1.0826x over previous
import functools

import jax
import jax.numpy as jnp
from jax import lax
from jax.experimental import pallas as pl
from jax.experimental.pallas import tpu as pltpu
from jax.experimental.pallas import tpu_sc as plsc

HIDDEN = 2048
NUM_EXPERTS = 16
TOP_K = 2


def _logits_body(x_ref, w_ref, out_ref):
    out_ref[...] = lax.dot_general(
        x_ref[...],
        w_ref[...],
        dimension_numbers=(((1,), (1,)), ((), ())),
        preferred_element_type=jnp.float32,
    )


def _logits(xf, W):
    n = xf.shape[0]
    bt = 2048
    return pl.pallas_call(
        _logits_body,
        grid=(n // bt,),
        in_specs=[
            pl.BlockSpec((bt, HIDDEN), lambda i: (i, 0)),
            pl.BlockSpec((NUM_EXPERTS, HIDDEN), lambda i: (0, 0)),
        ],
        out_specs=pl.BlockSpec((bt, NUM_EXPERTS), lambda i: (i, 0)),
        out_shape=jax.ShapeDtypeStruct((n, NUM_EXPERTS), jnp.float32),
    )(xf, W)


def _router(logits_flat, n):
    info = plsc.get_sparse_core_info()
    nw = info.num_cores * info.num_subcores
    tpw = n // nw
    mesh = plsc.VectorSubcoreMesh(core_axis_name="c", subcore_axis_name="s")

    @functools.partial(
        pl.kernel,
        out_type=(
            jax.ShapeDtypeStruct((n * NUM_EXPERTS,), jnp.float32),
            jax.ShapeDtypeStruct((n * TOP_K,), jnp.float32),
            jax.ShapeDtypeStruct((n * TOP_K,), jnp.int32),
        ),
        mesh=mesh,
        compiler_params=pltpu.CompilerParams(needs_layout_passes=False),
        scratch_types=[
            pltpu.VMEM((tpw * NUM_EXPERTS,), jnp.float32),
            pltpu.VMEM((tpw * NUM_EXPERTS,), jnp.float32),
            pltpu.VMEM((tpw * TOP_K,), jnp.float32),
            pltpu.VMEM((tpw * TOP_K,), jnp.int32),
        ],
    )
    def run(logits_hbm, scores_hbm, weights_hbm, inds_hbm, lg_v, sc_v, w_v, i_v):
        wid = lax.axis_index("s") * info.num_cores + lax.axis_index("c")
        base = wid * tpw
        pltpu.sync_copy(
            logits_hbm.at[pl.ds(base * NUM_EXPERTS, tpw * NUM_EXPERTS)], lg_v
        )
        lanes = lax.iota(jnp.int32, NUM_EXPERTS)

        def max_tree(v):
            for s in (1, 2, 4, 8):
                v = jnp.maximum(v, v.at[lanes ^ s].get(mode="promise_in_bounds"))
            return v

        @plsc.parallel_loop(0, tpw, unroll=8)
        def body(t):
            off = pl.multiple_of(t * NUM_EXPERTS, NUM_EXPERTS)
            row = lg_v[pl.ds(off, NUM_EXPERTS)]
            m1 = max_tree(row)
            i1 = plsc.all_reduce_ffs(row == m1)
            sel1 = lanes == i1
            row2 = jnp.where(sel1, -jnp.inf, row)
            m2 = max_tree(row2)
            i2 = plsc.all_reduce_ffs(row2 == m2)
            sel2 = lanes == i2
            chosen = sel1 | sel2
            sig = 1.0 / (1.0 + jnp.exp(-row))
            sc_v[pl.ds(off, NUM_EXPERTS)] = jnp.where(chosen, sig, 0.0)
            flat = jnp.where(sel1, TOP_K * t, TOP_K * t + 1)
            plsc.store_scatter(w_v, [flat], sig, mask=chosen)
            plsc.store_scatter(i_v, [flat], lanes, mask=chosen)
        pltpu.sync_copy(
            sc_v, scores_hbm.at[pl.ds(base * NUM_EXPERTS, tpw * NUM_EXPERTS)]
        )
        pltpu.sync_copy(w_v, weights_hbm.at[pl.ds(base * TOP_K, tpw * TOP_K)])
        pltpu.sync_copy(i_v, inds_hbm.at[pl.ds(base * TOP_K, tpw * TOP_K)])

    return run(logits_flat)


def kernel(x, W):
    xf = x.reshape(-1, x.shape[-1])
    n = xf.shape[0]
    logits = _logits(xf, W)
    scores, weights, inds = _router(logits.reshape(-1), n)
    return (
        scores.reshape(n, NUM_EXPERTS),
        weights.reshape(n, TOP_K),
        inds.reshape(n, TOP_K),
    )


# speedup vs baseline: 2.1290x; 1.0142x over previous
import functools

import jax
import jax.numpy as jnp
from jax import lax
from jax.experimental import pallas as pl
from jax.experimental.pallas import tpu as pltpu
from jax.experimental.pallas import tpu_sc as plsc

HIDDEN = 2048
NUM_EXPERTS = 16
TOP_K = 2


def _logits_body(x_ref, w_ref, out_ref):
    out_ref[...] = lax.dot_general(
        x_ref[...],
        w_ref[...],
        dimension_numbers=(((1,), (1,)), ((), ())),
        preferred_element_type=jnp.float32,
    )


def _logits(xf, W):
    n = xf.shape[0]
    bt = 1024
    return pl.pallas_call(
        _logits_body,
        grid=(n // bt,),
        in_specs=[
            pl.BlockSpec((bt, HIDDEN), lambda i: (i, 0)),
            pl.BlockSpec((NUM_EXPERTS, HIDDEN), lambda i: (0, 0)),
        ],
        out_specs=pl.BlockSpec((bt, NUM_EXPERTS), lambda i: (i, 0)),
        out_shape=jax.ShapeDtypeStruct((n, NUM_EXPERTS), jnp.float32),
        compiler_params=pltpu.CompilerParams(vmem_limit_bytes=100 * 2**20),
    )(xf, W)


def _router(logits_flat, n):
    info = plsc.get_sparse_core_info()
    nw = info.num_cores * info.num_subcores
    tpw = n // nw
    mesh = plsc.VectorSubcoreMesh(core_axis_name="c", subcore_axis_name="s")

    @functools.partial(
        pl.kernel,
        out_type=(
            jax.ShapeDtypeStruct((n * NUM_EXPERTS,), jnp.float32),
            jax.ShapeDtypeStruct((n * TOP_K,), jnp.float32),
            jax.ShapeDtypeStruct((n * TOP_K,), jnp.int32),
        ),
        mesh=mesh,
        compiler_params=pltpu.CompilerParams(needs_layout_passes=False),
        scratch_types=[
            pltpu.VMEM((tpw * NUM_EXPERTS,), jnp.float32),
            pltpu.VMEM((tpw * NUM_EXPERTS,), jnp.float32),
            pltpu.VMEM((tpw * TOP_K,), jnp.float32),
            pltpu.VMEM((tpw * TOP_K,), jnp.int32),
        ],
    )
    def run(logits_hbm, scores_hbm, weights_hbm, inds_hbm, lg_v, sc_v, w_v, i_v):
        wid = lax.axis_index("s") * info.num_cores + lax.axis_index("c")
        base = wid * tpw
        pltpu.sync_copy(
            logits_hbm.at[pl.ds(base * NUM_EXPERTS, tpw * NUM_EXPERTS)], lg_v
        )
        lanes = lax.iota(jnp.int32, NUM_EXPERTS)

        def max_tree(v):
            for s in (1, 2, 4, 8):
                v = jnp.maximum(v, v.at[lanes ^ s].get(mode="promise_in_bounds"))
            return v

        @plsc.parallel_loop(0, tpw, unroll=8)
        def body(t):
            off = pl.multiple_of(t * NUM_EXPERTS, NUM_EXPERTS)
            row = lg_v[pl.ds(off, NUM_EXPERTS)]
            m1 = max_tree(row)
            i1 = plsc.all_reduce_ffs(row == m1)
            sel1 = lanes == i1
            row2 = jnp.where(sel1, -jnp.inf, row)
            m2 = max_tree(row2)
            i2 = plsc.all_reduce_ffs(row2 == m2)
            sel2 = lanes == i2
            chosen = sel1 | sel2
            sig = 1.0 / (1.0 + jnp.exp(-row))
            sc_v[pl.ds(off, NUM_EXPERTS)] = jnp.where(chosen, sig, 0.0)
            flat = jnp.where(sel1, TOP_K * t, TOP_K * t + 1)
            plsc.store_scatter(w_v, [flat], sig, mask=chosen)
            plsc.store_scatter(i_v, [flat], lanes, mask=chosen)
        pltpu.sync_copy(
            sc_v, scores_hbm.at[pl.ds(base * NUM_EXPERTS, tpw * NUM_EXPERTS)]
        )
        pltpu.sync_copy(w_v, weights_hbm.at[pl.ds(base * TOP_K, tpw * TOP_K)])
        pltpu.sync_copy(i_v, inds_hbm.at[pl.ds(base * TOP_K, tpw * TOP_K)])

    return run(logits_flat)


def kernel(x, W):
    xf = x.reshape(-1, x.shape[-1])
    n = xf.shape[0]
    logits = _logits(xf, W)
    scores, weights, inds = _router(logits.reshape(-1), n)
    return (
        scores.reshape(n, NUM_EXPERTS),
        weights.reshape(n, TOP_K),
        inds.reshape(n, TOP_K),
    )
